# baseline (speedup 1.0000x reference)
import functools

import jax
import jax.numpy as jnp
from jax import lax
from jax.experimental import pallas as pl
from jax.experimental.pallas import tpu as pltpu
from jax.experimental.pallas import tpu_sc as plsc

_NC = 2
_NS = 16
_NQ = 4
_CHUNK = 128
_LANES = 16


@functools.lru_cache(maxsize=None)
def _make_spmm2(M_pad, CQ, nchunk):
    rows_per_tile = M_pad // _NS

    mesh = plsc.VectorSubcoreMesh(core_axis_name="c", subcore_axis_name="s")

    @functools.partial(
        pl.kernel,
        out_type=jax.ShapeDtypeStruct((_NQ, M_pad, CQ), jnp.float32),
        mesh=mesh,
        scratch_types=[
            pltpu.VMEM((nchunk, _CHUNK), jnp.int32),
            pltpu.VMEM((nchunk, _CHUNK), jnp.int32),
            pltpu.VMEM((nchunk, _CHUNK), jnp.float32),
            pltpu.VMEM((_CHUNK, CQ), jnp.float32),
            pltpu.VMEM((_CHUNK, CQ), jnp.float32),
            pltpu.VMEM((_CHUNK, CQ), jnp.float32),
            pltpu.VMEM((_CHUNK, CQ), jnp.float32),
            pltpu.VMEM_SHARED((M_pad, CQ), jnp.float32),
            pltpu.VMEM_SHARED((M_pad, CQ), jnp.float32),
            pltpu.SemaphoreType.DMA,
            pltpu.SemaphoreType.DMA,
            pltpu.SemaphoreType.DMA,
            pltpu.SemaphoreType.DMA,
        ],
        compiler_params=pltpu.CompilerParams(use_tc_tiling_on_sc=False),
    )
    def spmm2(t1_hbm, t0_hbm, rows_hbm, cols_hbm, vals_hbm, zinit_hbm,
              z_hbm, rows_v, cols_v, vals_v, g0, g1, s0, s1,
              xsrc, yacc, sg0, sg1, ss0, ss1):
        c = lax.axis_index("c")
        s = lax.axis_index("s")

        pltpu.sync_copy(rows_hbm.at[s], rows_v)
        pltpu.sync_copy(cols_hbm.at[s], cols_v)
        pltpu.sync_copy(vals_hbm.at[s], vals_v)
        base = pl.multiple_of(s * rows_per_tile, 8)
        stripe = pl.ds(base, rows_per_tile)

        def g_start(src, j, buf, sem):
            pltpu.async_copy(src.at[cols_v.at[j]], buf, sem)

        def g_wait(src, j, buf, sem):
            pltpu.make_async_copy(src.at[cols_v.at[j]], buf, sem).wait()

        def s_start(dst, j, buf, sem):
            pltpu.async_copy(buf, dst.at[rows_v.at[j]], sem, add=True)

        def s_wait(dst, j, buf, sem):
            pltpu.make_async_copy(buf, dst.at[rows_v.at[j]], sem).wait()

        def scale(j, gb, sb):
            for eg in range(_CHUNK // _LANES):
                v16 = vals_v[j, pl.ds(eg * _LANES, _LANES)]
                for l in range(_LANES):
                    e = eg * _LANES + l
                    v = jnp.broadcast_to(v16[l], (_LANES,))
                    for g in range(CQ // _LANES):
                        sl = pl.ds(g * _LANES, _LANES)
                        sb[e, sl] = gb[e, sl] * v

        assert nchunk % 2 == 0 and nchunk >= 4
        niter = nchunk // 2

        def edge_pass(src, dst):
            g_start(src, 0, g0, sg0)
            g_start(src, 1, g1, sg1)
            g_wait(src, 0, g0, sg0)
            scale(0, g0, s0)
            s_start(dst, 0, s0, ss0)
            g_start(src, 2, g0, sg0)
            g_wait(src, 1, g1, sg1)
            scale(1, g1, s1)
            s_start(dst, 1, s1, ss1)
            g_start(src, 3, g1, sg1)

            def pair_body(i, carry):
                j0 = 2 * i
                g_wait(src, j0, g0, sg0)
                s_wait(dst, j0 - 2, s0, ss0)
                scale(j0, g0, s0)
                s_start(dst, j0, s0, ss0)
                g_start(src, j0 + 2, g0, sg0)
                g_wait(src, j0 + 1, g1, sg1)
                s_wait(dst, j0 - 1, s1, ss1)
                scale(j0 + 1, g1, s1)
                s_start(dst, j0 + 1, s1, ss1)
                g_start(src, j0 + 3, g1, sg1)
                return carry

            lax.fori_loop(1, niter - 1, pair_body, 0)
            j0 = nchunk - 2
            g_wait(src, j0, g0, sg0)
            s_wait(dst, j0 - 2, s0, ss0)
            scale(j0, g0, s0)
            s_start(dst, j0, s0, ss0)
            g_wait(src, j0 + 1, g1, sg1)
            s_wait(dst, j0 - 1, s1, ss1)
            scale(j0 + 1, g1, s1)
            s_start(dst, j0 + 1, s1, ss1)
            s_wait(dst, j0, s0, ss0)
            s_wait(dst, j0 + 1, s1, ss1)

        def quarter_body(q, qcarry):
            qi = c * 2 + q
            pltpu.sync_copy(t1_hbm.at[qi, stripe], xsrc.at[stripe])
            pltpu.sync_copy(zinit_hbm, yacc.at[stripe])
            plsc.subcore_barrier()

            edge_pass(xsrc, yacc)

            plsc.subcore_barrier()

            def add_piece(off, n):
                psl = pl.ds(off, n)
                pltpu.sync_copy(t0_hbm.at[qi, psl], g0.at[pl.ds(0, n)])
                pltpu.sync_copy(yacc.at[psl], s0.at[pl.ds(0, n)])
                for r in range(n):
                    for g in range(CQ // _LANES):
                        sl = pl.ds(g * _LANES, _LANES)
                        s0[r, sl] = s0[r, sl] + g0[r, sl]
                pltpu.sync_copy(s0.at[pl.ds(0, n)], yacc.at[psl])

            n_full = rows_per_tile // _CHUNK
            tail = rows_per_tile - n_full * _CHUNK

            def piece_body(p, carry):
                add_piece(base + p * _CHUNK, _CHUNK)
                return carry

            lax.fori_loop(0, n_full, piece_body, 0)
            if tail:
                add_piece(base + n_full * _CHUNK, tail)
            pltpu.sync_copy(zinit_hbm, xsrc.at[stripe])
            plsc.subcore_barrier()

            edge_pass(yacc, xsrc)

            plsc.subcore_barrier()
            pltpu.sync_copy(xsrc.at[stripe], z_hbm.at[qi, stripe])
            return qcarry

        lax.fori_loop(0, 2, quarter_body, 0)

    return spmm2


def _pre(M_pad, CQ, x0, w0, w1):
    M = x0.shape[1]

    def body(x_ref, w0_ref, w1_ref, t0_ref, t1_ref):
        xm = x_ref[...]
        dn = (((0,), (1,)), ((), ()))
        for w_ref, t_ref in ((w0_ref, t0_ref), (w1_ref, t1_ref)):
            t = lax.dot_general(xm, w_ref[...], dn,
                                preferred_element_type=jnp.float32)
            for q in range(_NQ):
                t_ref[q, :M, :] = t[:, q * CQ:(q + 1) * CQ]
                t_ref[q, M:, :] = jnp.zeros((M_pad - M, CQ), jnp.float32)

    out = jax.ShapeDtypeStruct((_NQ, M_pad, CQ), jnp.float32)
    return pl.pallas_call(body, out_shape=(out, out))(x0, w0, w1)


def _post(M, zq, eye, bias_col):
    C = eye.shape[0]
    CQ = zq.shape[2]

    def body(z_ref, eye_ref, b_ref, o_ref):
        dn = (((1,), (1,)), ((), ()))
        acc = None
        for q in range(_NQ):
            t = lax.dot_general(eye_ref[:, q * CQ:(q + 1) * CQ], z_ref[q],
                                dn, preferred_element_type=jnp.float32)
            acc = t if acc is None else acc + t
        o_ref[0] = acc[:, :M] + b_ref[...]

    return pl.pallas_call(
        body, out_shape=jax.ShapeDtypeStruct((1, C, M), jnp.float32),
    )(zq, eye, bias_col)


def kernel(x, edge_index, edge_values, theta, bias):
    _, C_in, M = x.shape
    E = edge_index.shape[1]
    per = _NS * _CHUNK
    nchunk = -(-E // per)
    nchunk += nchunk % 2
    E_pad = nchunk * per

    rows = edge_index[0]
    cols = edge_index[1]
    vals = edge_values
    if E_pad != E:
        rows = jnp.pad(rows, (0, E_pad - E))
        cols = jnp.pad(cols, (0, E_pad - E))
        vals = jnp.pad(vals, (0, E_pad - E))
    rows3 = rows.reshape(_NS, nchunk, _CHUNK)
    cols3 = cols.reshape(_NS, nchunk, _CHUNK)
    vals3 = vals.reshape(_NS, nchunk, _CHUNK)

    rpt = (-(-M // _NS) + 7) // 8 * 8
    M_pad = rpt * _NS
    CQ = C_in // _NQ

    zinit = jnp.zeros((rpt, CQ), jnp.float32)
    eye = jnp.eye(C_in, dtype=jnp.float32)

    t0q, t1q = _pre(M_pad, CQ, x[0], theta[:, :, 0], theta[:, :, 1])
    spmm2 = _make_spmm2(M_pad, CQ, nchunk)
    zq = spmm2(t1q, t0q, rows3, cols3, vals3, zinit)
    return _post(M, zq, eye, bias[0])


# speedup vs baseline: 1.1353x; 1.1353x over previous
import functools

import jax
import jax.numpy as jnp
from jax import lax
from jax.experimental import pallas as pl
from jax.experimental.pallas import tpu as pltpu
from jax.experimental.pallas import tpu_sc as plsc

_NC = 2
_NS = 16
_NQ = 4
_CHUNK = 128
_LANES = 16


@functools.lru_cache(maxsize=None)
def _make_spmm2(M_pad, CQ, nchunk):
    rows_per_tile = M_pad // _NS

    mesh = plsc.VectorSubcoreMesh(core_axis_name="c", subcore_axis_name="s")

    @functools.partial(
        pl.kernel,
        out_type=jax.ShapeDtypeStruct((M_pad, _NQ * CQ), jnp.float32),
        mesh=mesh,
        scratch_types=[
            pltpu.VMEM((nchunk, _CHUNK), jnp.int32),
            pltpu.VMEM((nchunk, _CHUNK), jnp.int32),
            pltpu.VMEM((nchunk, _CHUNK), jnp.float32),
            pltpu.VMEM((_CHUNK, CQ), jnp.float32),
            pltpu.VMEM((_CHUNK, CQ), jnp.float32),
            pltpu.VMEM((_CHUNK, CQ), jnp.float32),
            pltpu.VMEM((_CHUNK, CQ), jnp.float32),
            pltpu.VMEM_SHARED((M_pad, CQ), jnp.float32),
            pltpu.VMEM_SHARED((M_pad, CQ), jnp.float32),
            pltpu.SemaphoreType.DMA,
            pltpu.SemaphoreType.DMA,
            pltpu.SemaphoreType.DMA,
            pltpu.SemaphoreType.DMA,
        ],
        compiler_params=pltpu.CompilerParams(use_tc_tiling_on_sc=False),
    )
    def spmm2(t1_hbm, t0_hbm, rows_hbm, cols_hbm, vals_hbm, zinit_hbm,
              z_hbm, rows_v, cols_v, vals_v, g0, g1, s0, s1,
              xsrc, yacc, sg0, sg1, ss0, ss1):
        c = lax.axis_index("c")
        s = lax.axis_index("s")

        pltpu.sync_copy(rows_hbm.at[s], rows_v)
        pltpu.sync_copy(cols_hbm.at[s], cols_v)
        pltpu.sync_copy(vals_hbm.at[s], vals_v)
        base = pl.multiple_of(s * rows_per_tile, 8)
        stripe = pl.ds(base, rows_per_tile)

        def g_start(src, j, buf, sem):
            pltpu.async_copy(src.at[cols_v.at[j]], buf, sem)

        def g_wait(src, j, buf, sem):
            pltpu.make_async_copy(src.at[cols_v.at[j]], buf, sem).wait()

        def s_start(dst, j, buf, sem):
            pltpu.async_copy(buf, dst.at[rows_v.at[j]], sem, add=True)

        def s_wait(dst, j, buf, sem):
            pltpu.make_async_copy(buf, dst.at[rows_v.at[j]], sem).wait()

        def scale(j, gb, sb):
            for eg in range(_CHUNK // _LANES):
                v16 = vals_v[j, pl.ds(eg * _LANES, _LANES)]
                for l in range(_LANES):
                    e = eg * _LANES + l
                    v = jnp.broadcast_to(v16[l], (_LANES,))
                    for g in range(CQ // _LANES):
                        sl = pl.ds(g * _LANES, _LANES)
                        sb[e, sl] = gb[e, sl] * v

        assert nchunk % 2 == 0 and nchunk >= 4
        niter = nchunk // 2

        def edge_pass(src, dst):
            g_start(src, 0, g0, sg0)
            g_start(src, 1, g1, sg1)
            g_wait(src, 0, g0, sg0)
            scale(0, g0, s0)
            s_start(dst, 0, s0, ss0)
            g_start(src, 2, g0, sg0)
            g_wait(src, 1, g1, sg1)
            scale(1, g1, s1)
            s_start(dst, 1, s1, ss1)
            g_start(src, 3, g1, sg1)

            def pair_body(i, carry):
                j0 = 2 * i
                g_wait(src, j0, g0, sg0)
                s_wait(dst, j0 - 2, s0, ss0)
                scale(j0, g0, s0)
                s_start(dst, j0, s0, ss0)
                g_start(src, j0 + 2, g0, sg0)
                g_wait(src, j0 + 1, g1, sg1)
                s_wait(dst, j0 - 1, s1, ss1)
                scale(j0 + 1, g1, s1)
                s_start(dst, j0 + 1, s1, ss1)
                g_start(src, j0 + 3, g1, sg1)
                return carry

            lax.fori_loop(1, niter - 1, pair_body, 0)
            j0 = nchunk - 2
            g_wait(src, j0, g0, sg0)
            s_wait(dst, j0 - 2, s0, ss0)
            scale(j0, g0, s0)
            s_start(dst, j0, s0, ss0)
            g_wait(src, j0 + 1, g1, sg1)
            s_wait(dst, j0 - 1, s1, ss1)
            scale(j0 + 1, g1, s1)
            s_start(dst, j0 + 1, s1, ss1)
            s_wait(dst, j0, s0, ss0)
            s_wait(dst, j0 + 1, s1, ss1)

        def quarter_body(q, qcarry):
            qi = c * 2 + q
            qsl = pl.ds(pl.multiple_of(qi * CQ, 8), CQ)
            pltpu.sync_copy(t1_hbm.at[stripe, qsl], xsrc.at[stripe])
            pltpu.sync_copy(zinit_hbm, yacc.at[stripe])
            plsc.subcore_barrier()

            edge_pass(xsrc, yacc)

            plsc.subcore_barrier()

            def add_piece(off, n):
                psl = pl.ds(off, n)
                pltpu.sync_copy(t0_hbm.at[psl, qsl], g0.at[pl.ds(0, n)])
                pltpu.sync_copy(yacc.at[psl], s0.at[pl.ds(0, n)])
                for r in range(n):
                    for g in range(CQ // _LANES):
                        sl = pl.ds(g * _LANES, _LANES)
                        s0[r, sl] = s0[r, sl] + g0[r, sl]
                pltpu.sync_copy(s0.at[pl.ds(0, n)], yacc.at[psl])

            n_full = rows_per_tile // _CHUNK
            tail = rows_per_tile - n_full * _CHUNK

            def piece_body(p, carry):
                add_piece(base + p * _CHUNK, _CHUNK)
                return carry

            lax.fori_loop(0, n_full, piece_body, 0)
            if tail:
                add_piece(base + n_full * _CHUNK, tail)
            pltpu.sync_copy(zinit_hbm, xsrc.at[stripe])
            plsc.subcore_barrier()

            edge_pass(yacc, xsrc)

            plsc.subcore_barrier()
            pltpu.sync_copy(xsrc.at[stripe], z_hbm.at[stripe, qsl])
            return qcarry

        lax.fori_loop(0, 2, quarter_body, 0)

    return spmm2


def _pre(M_pad, x0, w0, w1):
    M = x0.shape[1]
    C = w0.shape[0]

    def body(x_ref, w0_ref, w1_ref, t0_ref, t1_ref):
        xm = x_ref[...]
        dn = (((0,), (1,)), ((), ()))
        for w_ref, t_ref in ((w0_ref, t0_ref), (w1_ref, t1_ref)):
            t = lax.dot_general(xm, w_ref[...], dn,
                                preferred_element_type=jnp.float32)
            t_ref[:M, :] = t
            t_ref[M:, :] = jnp.zeros((M_pad - M, C), jnp.float32)

    out = jax.ShapeDtypeStruct((M_pad, C), jnp.float32)
    return pl.pallas_call(body, out_shape=(out, out))(x0, w0, w1)


def _post(M, z, eye, bias_col):
    C = eye.shape[0]

    def body(z_ref, eye_ref, b_ref, o_ref):
        dn = (((1,), (1,)), ((), ()))
        acc = lax.dot_general(eye_ref[...], z_ref[...], dn,
                              preferred_element_type=jnp.float32)
        o_ref[0] = acc[:, :M] + b_ref[...]

    return pl.pallas_call(
        body, out_shape=jax.ShapeDtypeStruct((1, C, M), jnp.float32),
    )(z, eye, bias_col)


def kernel(x, edge_index, edge_values, theta, bias):
    _, C_in, M = x.shape
    E = edge_index.shape[1]
    per = _NS * _CHUNK
    nchunk = (-(-E // per) + 7) // 8 * 8
    E_pad = nchunk * per

    rows = edge_index[0]
    cols = edge_index[1]
    vals = edge_values
    if E_pad != E:
        rows = jnp.pad(rows, (0, E_pad - E))
        cols = jnp.pad(cols, (0, E_pad - E))
        vals = jnp.pad(vals, (0, E_pad - E))
    rows3 = rows.reshape(_NS, nchunk, _CHUNK)
    cols3 = cols.reshape(_NS, nchunk, _CHUNK)
    vals3 = vals.reshape(_NS, nchunk, _CHUNK)

    rpt = (-(-M // _NS) + 7) // 8 * 8
    M_pad = rpt * _NS
    CQ = C_in // _NQ

    zinit = jnp.zeros((rpt, CQ), jnp.float32)
    eye = jnp.eye(C_in, dtype=jnp.float32)

    t0, t1 = _pre(M_pad, x[0], theta[:, :, 0], theta[:, :, 1])
    spmm2 = _make_spmm2(M_pad, CQ, nchunk)
    z = spmm2(t1, t0, rows3, cols3, vals3, zinit)
    return _post(M, z, eye, bias[0])
